# baseline (speedup 1.0000x reference)
import jax
import jax.numpy as jnp
from jax import lax
from jax.experimental import pallas as pl
from jax.experimental.pallas import tpu as pltpu

V = 100000
D = 10
H = 128
L = 200

RBP = 5120
NBP = (V + RBP - 1) // RBP
TAIL = V - (NBP - 1) * RBP


def _fused_body(idx_ref, tbl_ref, w1t_ref, b1_ref, w2_ref, b2_ref,
                out_ref, h_ref, m_ref, s_ref, tiles_ref, dma_sem):
    j = pl.program_id(0)

    @pl.when(j == 0)
    def _():
        def mk_copy(i):
            v = idx_ref[i]
            base = pl.multiple_of((v >> 7) * 128, 128)
            return pltpu.make_async_copy(
                tbl_ref.at[:, pl.ds(base, 128)],
                tiles_ref.at[:, pl.ds(i * 128, 128)],
                dma_sem,
            )

        lax.fori_loop(0, L, lambda i, c: (mk_copy(i).start(), c)[1], 0)
        lax.fori_loop(0, L, lambda i, c: (mk_copy(i).wait(), c)[1], 0)

        col_iota = lax.broadcasted_iota(jnp.int32, (D, 128), 1)

        def body(i, acc):
            v = idx_ref[i]
            c = v & 127
            tile = tiles_ref[:, pl.ds(i * 128, 128)]
            ssv = jnp.sum(tile * tile, axis=0, keepdims=True)
            scale = jnp.where(ssv > 1.0, lax.rsqrt(ssv), 1.0)
            return acc + jnp.where(col_iota == c, tile * scale, 0.0)

        acc = lax.fori_loop(0, L, body, jnp.zeros((D, 128), jnp.float32))
        x = jnp.sum(acc, axis=1, keepdims=True)
        h = lax.dot_general(x, w1t_ref[...], (((0,), (0,)), ((), ())),
                            preferred_element_type=jnp.float32)
        h_ref[...] = jnp.maximum(h + b1_ref[...], 0.0)

    h = h_ref[...]
    logits = lax.dot_general(h, w2_ref[...], (((1,), (1,)), ((), ())),
                             preferred_element_type=jnp.float32)
    logits = logits + b2_ref[...]

    valid = (lax.broadcasted_iota(jnp.int32, (1, RBP), 1) + j * RBP) < V
    lm = jnp.where(valid, logits, -1e30)

    base = pl.multiple_of(j * RBP, 128)

    @pl.when(j < NBP - 1)
    def _():
        out_ref[0, pl.ds(base, RBP)] = logits[0]

    @pl.when(j == NBP - 1)
    def _():
        out_ref[0, pl.ds(base, TAIL)] = logits[0, :TAIL]

    bm = jnp.max(lm, axis=(0, 1), keepdims=True)

    @pl.when(j == 0)
    def _():
        m_ref[...] = bm
        s_ref[...] = jnp.sum(jnp.exp(lm - bm), axis=(0, 1), keepdims=True)

    @pl.when(j > 0)
    def _():
        m_old = m_ref[...]
        nm = jnp.maximum(m_old, bm)
        s_ref[...] = (s_ref[...] * jnp.exp(m_old - nm)
                      + jnp.sum(jnp.exp(lm - nm), axis=(0, 1), keepdims=True))
        m_ref[...] = nm

    @pl.when(j == NBP - 1)
    def _():
        lse = m_ref[0, 0] + jnp.log(s_ref[0, 0])
        out_ref[...] = out_ref[...] - lse


def _make(interpret=False):
    return pl.pallas_call(
        _fused_body,
        grid=(NBP,),
        in_specs=[
            pl.BlockSpec(memory_space=pltpu.SMEM),
            pl.BlockSpec(memory_space=pl.ANY),
            pl.BlockSpec((D, H), lambda j: (0, 0)),
            pl.BlockSpec((1, H), lambda j: (0, 0)),
            pl.BlockSpec((RBP, H), lambda j: (j, 0)),
            pl.BlockSpec((1, RBP), lambda j: (0, j)),
        ],
        out_specs=pl.BlockSpec((1, V), lambda j: (0, 0)),
        out_shape=jax.ShapeDtypeStruct((1, V), jnp.float32),
        scratch_shapes=[
            pltpu.VMEM((1, H), jnp.float32),
            pltpu.VMEM((1, 1), jnp.float32),
            pltpu.VMEM((1, 1), jnp.float32),
            pltpu.VMEM((D, L * 128), jnp.float32),
            pltpu.SemaphoreType.DMA,
        ],
        interpret=interpret,
    )


def kernel(inputs, table, W1, b1, W2, b2):
    return _make()(
        inputs,
        table.T,
        W1.T,
        b1.reshape(1, H),
        W2,
        b2.reshape(1, V),
    )


# speedup vs baseline: 1.2666x; 1.2666x over previous
import jax
import jax.numpy as jnp
from jax import lax
from jax.experimental import pallas as pl
from jax.experimental.pallas import tpu as pltpu

V = 100000
D = 10
H = 128
L = 200

RBP = 10240
NBP = (V + RBP - 1) // RBP
TAIL = V - (NBP - 1) * RBP


def _fused_body(idx_ref, tbl_ref, w1t_ref, b1_ref, w2_ref, b2_ref,
                out_ref, h_ref, m_ref, s_ref):
    j = pl.program_id(0)

    @pl.when(j == 0)
    def _():
        col_iota = lax.broadcasted_iota(jnp.int32, (D, 128), 1)

        def body(i, acc):
            v = idx_ref[i]
            base = pl.multiple_of((v >> 7) * 128, 128)
            c = v & 127
            tile = tbl_ref[:, pl.ds(base, 128)]
            ssv = jnp.sum(tile * tile, axis=0, keepdims=True)
            scale = jnp.where(ssv > 1.0, lax.rsqrt(ssv), 1.0)
            return acc + jnp.where(col_iota == c, tile * scale, 0.0)

        acc = lax.fori_loop(0, L, body, jnp.zeros((D, 128), jnp.float32))
        x = jnp.sum(acc, axis=1, keepdims=True)
        h = lax.dot_general(x, w1t_ref[...], (((0,), (0,)), ((), ())),
                            preferred_element_type=jnp.float32)
        h_ref[...] = jnp.maximum(h + b1_ref[...], 0.0)

    h = h_ref[...]
    logits = lax.dot_general(h, w2_ref[...], (((1,), (1,)), ((), ())),
                             preferred_element_type=jnp.float32)
    logits = logits + b2_ref[...]

    valid = (lax.broadcasted_iota(jnp.int32, (1, RBP), 1) + j * RBP) < V
    lm = jnp.where(valid, logits, -1e30)

    base = pl.multiple_of(j * RBP, 128)

    @pl.when(j < NBP - 1)
    def _():
        out_ref[0, pl.ds(base, RBP)] = logits[0]

    @pl.when(j == NBP - 1)
    def _():
        out_ref[0, pl.ds(base, TAIL)] = logits[0, :TAIL]

    bm = jnp.max(lm, axis=(0, 1), keepdims=True)

    @pl.when(j == 0)
    def _():
        m_ref[...] = bm
        s_ref[...] = jnp.sum(jnp.exp(lm - bm), axis=(0, 1), keepdims=True)

    @pl.when(j > 0)
    def _():
        m_old = m_ref[...]
        nm = jnp.maximum(m_old, bm)
        s_ref[...] = (s_ref[...] * jnp.exp(m_old - nm)
                      + jnp.sum(jnp.exp(lm - nm), axis=(0, 1), keepdims=True))
        m_ref[...] = nm

    @pl.when(j == NBP - 1)
    def _():
        lse = m_ref[0, 0] + jnp.log(s_ref[0, 0])
        out_ref[...] = out_ref[...] - lse


def _make(interpret=False):
    return pl.pallas_call(
        _fused_body,
        grid=(NBP,),
        in_specs=[
            pl.BlockSpec(memory_space=pltpu.SMEM),
            pl.BlockSpec((D, V), lambda j: (0, 0)),
            pl.BlockSpec((D, H), lambda j: (0, 0)),
            pl.BlockSpec((1, H), lambda j: (0, 0)),
            pl.BlockSpec((RBP, H), lambda j: (j, 0)),
            pl.BlockSpec((1, RBP), lambda j: (0, j)),
        ],
        out_specs=pl.BlockSpec((1, V), lambda j: (0, 0)),
        out_shape=jax.ShapeDtypeStruct((1, V), jnp.float32),
        scratch_shapes=[
            pltpu.VMEM((1, H), jnp.float32),
            pltpu.VMEM((1, 1), jnp.float32),
            pltpu.VMEM((1, 1), jnp.float32),
        ],
        interpret=interpret,
    )


def kernel(inputs, table, W1, b1, W2, b2):
    return _make()(
        inputs,
        table.T,
        W1.T,
        b1.reshape(1, H),
        W2,
        b2.reshape(1, V),
    )


# speedup vs baseline: 1.3707x; 1.0822x over previous
import jax
import jax.numpy as jnp
from jax import lax
from jax.experimental import pallas as pl
from jax.experimental.pallas import tpu as pltpu

V = 100000
D = 10
H = 128
L = 200

RBP = 20480
NBP = (V + RBP - 1) // RBP
TAIL = V - (NBP - 1) * RBP


def _fused_body(idx_ref, tbl_ref, w1t_ref, b1_ref, w2_ref, b2_ref,
                out_ref, h_ref, m_ref, s_ref):
    j = pl.program_id(0)

    @pl.when(j == 0)
    def _():
        col_iota = lax.broadcasted_iota(jnp.int32, (D, 128), 1)

        def body(i, acc):
            v = idx_ref[i]
            base = pl.multiple_of((v >> 7) * 128, 128)
            c = v & 127
            tile = tbl_ref[:, pl.ds(base, 128)]
            ssv = jnp.sum(tile * tile, axis=0, keepdims=True)
            scale = jnp.where(ssv > 1.0, lax.rsqrt(ssv), 1.0)
            return acc + jnp.where(col_iota == c, tile * scale, 0.0)

        acc = lax.fori_loop(0, L, body, jnp.zeros((D, 128), jnp.float32))
        x = jnp.sum(acc, axis=1, keepdims=True)
        h = lax.dot_general(x, w1t_ref[...], (((0,), (0,)), ((), ())),
                            preferred_element_type=jnp.float32)
        h_ref[...] = jnp.maximum(h + b1_ref[...], 0.0)

    h = h_ref[...]
    logits = lax.dot_general(h, w2_ref[...], (((1,), (1,)), ((), ())),
                             preferred_element_type=jnp.float32)
    logits = logits + b2_ref[...]

    valid = (lax.broadcasted_iota(jnp.int32, (1, RBP), 1) + j * RBP) < V
    lm = jnp.where(valid, logits, -1e30)

    base = pl.multiple_of(j * RBP, 128)

    @pl.when(j < NBP - 1)
    def _():
        out_ref[0, pl.ds(base, RBP)] = logits[0]

    @pl.when(j == NBP - 1)
    def _():
        out_ref[0, pl.ds(base, TAIL)] = logits[0, :TAIL]

    bm = jnp.max(lm, axis=(0, 1), keepdims=True)

    @pl.when(j == 0)
    def _():
        m_ref[...] = bm
        s_ref[...] = jnp.sum(jnp.exp(lm - bm), axis=(0, 1), keepdims=True)

    @pl.when(j > 0)
    def _():
        m_old = m_ref[...]
        nm = jnp.maximum(m_old, bm)
        s_ref[...] = (s_ref[...] * jnp.exp(m_old - nm)
                      + jnp.sum(jnp.exp(lm - nm), axis=(0, 1), keepdims=True))
        m_ref[...] = nm

    @pl.when(j == NBP - 1)
    def _():
        lse = m_ref[0, 0] + jnp.log(s_ref[0, 0])
        out_ref[...] = out_ref[...] - lse


def _make(interpret=False):
    return pl.pallas_call(
        _fused_body,
        grid=(NBP,),
        in_specs=[
            pl.BlockSpec(memory_space=pltpu.SMEM),
            pl.BlockSpec((D, V), lambda j: (0, 0)),
            pl.BlockSpec((D, H), lambda j: (0, 0)),
            pl.BlockSpec((1, H), lambda j: (0, 0)),
            pl.BlockSpec((RBP, H), lambda j: (j, 0)),
            pl.BlockSpec((1, RBP), lambda j: (0, j)),
        ],
        out_specs=pl.BlockSpec((1, V), lambda j: (0, 0)),
        out_shape=jax.ShapeDtypeStruct((1, V), jnp.float32),
        scratch_shapes=[
            pltpu.VMEM((1, H), jnp.float32),
            pltpu.VMEM((1, 1), jnp.float32),
            pltpu.VMEM((1, 1), jnp.float32),
        ],
        interpret=interpret,
    )


def kernel(inputs, table, W1, b1, W2, b2):
    return _make()(
        inputs,
        table.T,
        W1.T,
        b1.reshape(1, H),
        W2,
        b2.reshape(1, V),
    )


# speedup vs baseline: 1.4013x; 1.0223x over previous
import jax
import jax.numpy as jnp
from jax import lax
from jax.experimental import pallas as pl
from jax.experimental.pallas import tpu as pltpu

V = 100000
D = 10
H = 128
L = 200

RBP = 25600
NBP = (V + RBP - 1) // RBP
TAIL = V - (NBP - 1) * RBP


def _fused_body(idx_ref, tbl_ref, w1t_ref, b1_ref, w2_ref, b2_ref,
                out_ref, h_ref, m_ref, s_ref):
    j = pl.program_id(0)

    @pl.when(j == 0)
    def _():
        col_iota = lax.broadcasted_iota(jnp.int32, (D, 128), 1)

        def body(i, acc):
            v = idx_ref[i]
            base = pl.multiple_of((v >> 7) * 128, 128)
            c = v & 127
            tile = tbl_ref[:, pl.ds(base, 128)]
            ssv = jnp.sum(tile * tile, axis=0, keepdims=True)
            scale = jnp.where(ssv > 1.0, lax.rsqrt(ssv), 1.0)
            return acc + jnp.where(col_iota == c, tile * scale, 0.0)

        acc = lax.fori_loop(0, L, body, jnp.zeros((D, 128), jnp.float32))
        x = jnp.sum(acc, axis=1, keepdims=True)
        h = lax.dot_general(x, w1t_ref[...], (((0,), (0,)), ((), ())),
                            preferred_element_type=jnp.float32)
        h_ref[...] = jnp.maximum(h + b1_ref[...], 0.0)

    h = h_ref[...]
    logits = lax.dot_general(h, w2_ref[...], (((1,), (1,)), ((), ())),
                             preferred_element_type=jnp.float32)
    logits = logits + b2_ref[...]

    valid = (lax.broadcasted_iota(jnp.int32, (1, RBP), 1) + j * RBP) < V
    lm = jnp.where(valid, logits, -1e30)

    base = pl.multiple_of(j * RBP, 128)

    @pl.when(j < NBP - 1)
    def _():
        out_ref[0, pl.ds(base, RBP)] = logits[0]

    @pl.when(j == NBP - 1)
    def _():
        out_ref[0, pl.ds(base, TAIL)] = logits[0, :TAIL]

    bm = jnp.max(lm, axis=(0, 1), keepdims=True)

    @pl.when(j == 0)
    def _():
        m_ref[...] = bm
        s_ref[...] = jnp.sum(jnp.exp(lm - bm), axis=(0, 1), keepdims=True)

    @pl.when(j > 0)
    def _():
        m_old = m_ref[...]
        nm = jnp.maximum(m_old, bm)
        s_ref[...] = (s_ref[...] * jnp.exp(m_old - nm)
                      + jnp.sum(jnp.exp(lm - nm), axis=(0, 1), keepdims=True))
        m_ref[...] = nm

    @pl.when(j == NBP - 1)
    def _():
        lse = m_ref[0, 0] + jnp.log(s_ref[0, 0])
        out_ref[...] = out_ref[...] - lse


def _make(interpret=False):
    return pl.pallas_call(
        _fused_body,
        grid=(NBP,),
        in_specs=[
            pl.BlockSpec(memory_space=pltpu.SMEM),
            pl.BlockSpec((D, V), lambda j: (0, 0)),
            pl.BlockSpec((D, H), lambda j: (0, 0)),
            pl.BlockSpec((1, H), lambda j: (0, 0)),
            pl.BlockSpec((RBP, H), lambda j: (j, 0)),
            pl.BlockSpec((1, RBP), lambda j: (0, j)),
        ],
        out_specs=pl.BlockSpec((1, V), lambda j: (0, 0)),
        out_shape=jax.ShapeDtypeStruct((1, V), jnp.float32),
        scratch_shapes=[
            pltpu.VMEM((1, H), jnp.float32),
            pltpu.VMEM((1, 1), jnp.float32),
            pltpu.VMEM((1, 1), jnp.float32),
        ],
        interpret=interpret,
    )


def kernel(inputs, table, W1, b1, W2, b2):
    return _make()(
        inputs,
        table.T,
        W1.T,
        b1.reshape(1, H),
        W2,
        b2.reshape(1, V),
    )
